# baseline (speedup 1.0000x reference)
import functools

import jax
import jax.numpy as jnp
from jax import lax
from jax.experimental import pallas as pl
from jax.experimental.pallas import tpu as pltpu
from jax.experimental.pallas import tpu_sc as plsc

N = 10000
X = 128
H = 128
K = 48
NC = 2
NS = 16
NH = N // NC
ACC_ROWS = 5008
SB = 8
NSB = K // SB
BN = 1000


_mesh = plsc.VectorSubcoreMesh(core_axis_name="c", subcore_axis_name="s")


@functools.partial(
    pl.kernel,
    out_type=[
        jax.ShapeDtypeStruct((N, X), jnp.float32),
        jax.ShapeDtypeStruct((N, X), jnp.float32),
    ],
    mesh=_mesh,
    compiler_params=pltpu.CompilerParams(needs_layout_passes=False),
    scratch_types=[
        pltpu.VMEM((16,), jnp.int32),
        pltpu.VMEM((K,), jnp.int32),
        pltpu.VMEM((K,), jnp.int32),
        pltpu.VMEM((K,), jnp.int32),
        pltpu.VMEM((K,), jnp.int32),
        pltpu.VMEM((K,), jnp.float32),
        pltpu.VMEM((K,), jnp.float32),
        pltpu.VMEM((K,), jnp.float32),
        pltpu.VMEM((K,), jnp.float32),
        pltpu.VMEM((K, X), jnp.float32),
        pltpu.VMEM((K, X), jnp.float32),
        pltpu.VMEM((K, X), jnp.float32),
        pltpu.VMEM((K, X), jnp.float32),
        pltpu.VMEM((K, X), jnp.float32),
        pltpu.VMEM((K, X), jnp.float32),
        pltpu.VMEM((NSB, SB), jnp.int32),
        pltpu.VMEM((NSB, SB), jnp.int32),
        pltpu.SMEM((8,), jnp.int32),
        pltpu.VMEM_SHARED((ACC_ROWS, X), jnp.float32),
        pltpu.VMEM_SHARED((ACC_ROWS, X), jnp.float32),
        pltpu.SemaphoreType.DMA,
        pltpu.SemaphoreType.DMA,
        pltpu.SemaphoreType.DMA,
        pltpu.SemaphoreType.DMA,
        pltpu.SemaphoreType.DMA,
        pltpu.SemaphoreType.DMA,
    ],
)
def _sc_segment_sums(h_hbm, src_hbm, dst_hbm, cnt_hbm, st_hbm, bounds_hbm,
                     s_out, b_out,
                     brow_v, sidx0, sidx1, didx0, didx1,
                     cnt0, cnt1, st0, st1, rows0, rows1,
                     fbs0, fbs1, fbb0, fbb1, fidx0, fidx1, nseg_sm,
                     s_acc, b_acc,
                     sem_i0, sem_i1, sem_g0, sem_g1, sem_s0, sem_s1):
    c = lax.axis_index("c")
    s = lax.axis_index("s")
    wid = c * NS + s
    sidx = (sidx0, sidx1)
    didx = (didx0, didx1)
    cnt = (cnt0, cnt1)
    st = (st0, st1)
    rows = (rows0, rows1)
    fbs = (fbs0, fbs1)
    fbb = (fbb0, fbb1)
    fidx = (fidx0, fidx1)
    sem_i = (sem_i0, sem_i1)
    sem_g = (sem_g0, sem_g1)
    sem_s = (sem_s0, sem_s1)
    iota16 = lax.iota(jnp.int32, 16)
    zeros16 = jnp.zeros((16,), jnp.float32)
    lane0 = iota16 == 0

    for r in range(K):
        for j in range(X // 16):
            fbs0[r, pl.ds(j * 16, 16)] = zeros16
    NZS = ACC_ROWS // K
    for q in range(7):
        zidx = s * 7 + q

        @pl.when(zidx < NZS)
        def _():
            pltpu.async_copy(fbs0, s_acc.at[pl.ds(zidx * K, K)], sem_g0)
            pltpu.async_copy(fbs0, b_acc.at[pl.ds(zidx * K, K)], sem_g0)
    for q in range(7):
        zidx = s * 7 + q

        @pl.when(zidx < NZS)
        def _():
            pltpu.make_async_copy(h_hbm.at[pl.ds(0, K)], fbs0,
                                  sem_g0).wait()
            pltpu.make_async_copy(h_hbm.at[pl.ds(0, K)], fbs0,
                                  sem_g0).wait()

    @pl.when(s == 0)
    def _():
        pltpu.sync_copy(fbs0.at[pl.ds(0, ACC_ROWS - NZS * K)],
                        s_acc.at[pl.ds(NZS * K, ACC_ROWS - NZS * K)])
        pltpu.sync_copy(fbs0.at[pl.ds(0, ACC_ROWS - NZS * K)],
                        b_acc.at[pl.ds(NZS * K, ACC_ROWS - NZS * K)])

    nseg_sm[0] = 0
    nseg_sm[1] = 0
    plsc.subcore_barrier()

    pltpu.sync_copy(bounds_hbm.at[wid], brow_v)
    b16 = brow_v[...]
    a_lo = b16[0]
    t_lo = b16[1]
    t_hi = b16[2]
    nch = b16[3]

    def issue_idx(chunk, b):
        base = pl.multiple_of(a_lo + chunk * K, 8)
        pltpu.async_copy(src_hbm.at[pl.ds(base, K)], sidx[b], sem_i[b])
        pltpu.async_copy(dst_hbm.at[pl.ds(base, K)], didx[b], sem_i[b])

    def wait_idx(b):
        pltpu.make_async_copy(src_hbm.at[pl.ds(0, K)], sidx[b],
                              sem_i[b]).wait()
        pltpu.make_async_copy(dst_hbm.at[pl.ds(0, K)], didx[b],
                              sem_i[b]).wait()

    def issue_gathers(b):
        pltpu.async_copy(h_hbm.at[sidx[b]], rows[b], sem_g[b])
        pltpu.async_copy(cnt_hbm.at[didx[b]], cnt[b], sem_g[b])
        pltpu.async_copy(st_hbm.at[didx[b]], st[b], sem_g[b])

    def wait_gathers(b):
        pltpu.make_async_copy(h_hbm.at[pl.ds(0, K)], rows[b],
                              sem_g[b]).wait()
        pltpu.make_async_copy(cnt_hbm.at[pl.ds(0, K)], cnt[b],
                              sem_g[b]).wait()
        pltpu.make_async_copy(st_hbm.at[pl.ds(0, K)], st[b],
                              sem_g[b]).wait()

    def drain_scatters(b):
        nseg_old = nseg_sm[b]
        for sb in range(NSB):
            @pl.when(nseg_old > sb * SB)
            def _():
                pltpu.make_async_copy(fbs[b].at[pl.ds(sb * SB, SB)],
                                      s_acc.at[fidx[b].at[sb]],
                                      sem_s[b]).wait()
                pltpu.make_async_copy(fbb[b].at[pl.ds(sb * SB, SB)],
                                      b_acc.at[fidx[b].at[sb]],
                                      sem_s[b]).wait()

    @pl.when(nch >= 1)
    def _():
        issue_idx(0, 0)

    @pl.when(nch >= 2)
    def _():
        issue_idx(1, 1)

    @pl.when(nch >= 1)
    def _():
        wait_idx(0)
        issue_gathers(0)

    def pair_body(it, carry):
        for b in range(2):
            chunk = 2 * it + b

            @pl.when(chunk < nch)
            def _():
                base = pl.multiple_of(a_lo + chunk * K, 8)
                wait_gathers(b)
                drain_scatters(b)

                dls = []
                rws = []
                for g in range(K // 16):
                    evec = base + g * 16 + iota16
                    d16 = didx[b][pl.ds(g * 16, 16)]
                    valid = jnp.logical_and(evec >= t_lo, evec < t_hi)
                    dls.append(jnp.where(valid, d16 - c * NH, -1))
                    cnt16 = cnt[b][pl.ds(g * 16, 16)]
                    st16 = st[b][pl.ds(g * 16, 16)]
                    pos = evec.astype(jnp.float32) - st16
                    rws.append(pos / jnp.maximum(cnt16 - 1.0, 1.0))

                saccs = [zeros16] * (X // 16)
                baccs = [zeros16] * (X // 16)
                nseg = jnp.int32(0)
                for k in range(K):
                    g, l = divmod(k, 16)
                    d_k = dls[g][l]
                    rwb = jnp.full((16,), rws[g][l], jnp.float32)
                    for j in range(X // 16):
                        r_j = rows[b][k, pl.ds(j * 16, 16)]
                        saccs[j] = saccs[j] + r_j
                        baccs[j] = baccs[j] + rwb * r_j
                    flushrow = jnp.where(d_k < 0, NH, d_k)
                    i0 = jnp.full((16,), nseg, jnp.int32)
                    isb = jnp.full((16,), lax.shift_right_logical(nseg, 3),
                                   jnp.int32)
                    ilo = jnp.full((16,), lax.rem(nseg, SB), jnp.int32)

                    def flush(sa=tuple(saccs), ba=tuple(baccs), i0=i0,
                              isb=isb, ilo=ilo, fr=flushrow):
                        for j in range(X // 16):
                            i1 = j * 16 + iota16
                            plsc.store_scatter(fbs[b], [i0, i1], sa[j])
                            plsc.store_scatter(fbb[b], [i0, i1], ba[j])
                        plsc.store_scatter(
                            fidx[b], [isb, ilo],
                            jnp.full((16,), fr, jnp.int32), mask=lane0)

                    if k == K - 1:
                        flush()
                        nseg = nseg + 1
                    else:
                        gn, ln = divmod(k + 1, 16)
                        is_end = d_k != dls[gn][ln]
                        pl.when(is_end)(flush)
                        for j in range(X // 16):
                            saccs[j] = jnp.where(is_end, zeros16, saccs[j])
                            baccs[j] = jnp.where(is_end, zeros16, baccs[j])
                        nseg = nseg + jnp.where(is_end, 1, 0).astype(
                            jnp.int32)

                nsegb = jnp.full((16,), nseg, jnp.int32)
                for g in range(K // 16):
                    lin = iota16 + g * 16
                    isb = lax.shift_right_logical(lin, 3)
                    ilo = lax.rem(lin, SB)
                    f16 = plsc.load_gather(fidx[b], [isb, ilo])
                    f16 = jnp.where(lin >= nsegb, NH, f16)
                    plsc.store_scatter(fidx[b], [isb, ilo], f16)

                nseg_sm[b] = nseg

                for sb in range(NSB):
                    @pl.when(nseg > sb * SB)
                    def _():
                        pltpu.async_copy(fbs[b].at[pl.ds(sb * SB, SB)],
                                         s_acc.at[fidx[b].at[sb]],
                                         sem_s[b], add=True)
                        pltpu.async_copy(fbb[b].at[pl.ds(sb * SB, SB)],
                                         b_acc.at[fidx[b].at[sb]],
                                         sem_s[b], add=True)

                @pl.when(chunk + 2 < nch)
                def _():
                    issue_idx(chunk + 2, b)

                @pl.when(chunk + 1 < nch)
                def _():
                    wait_idx(1 - b)
                    issue_gathers(1 - b)

        return carry

    lax.fori_loop(0, (nch + 1) // 2, pair_body, 0)
    drain_scatters(0)
    drain_scatters(1)
    plsc.subcore_barrier()

    OR = 624

    @pl.when(s < 7)
    def _():
        off = pl.multiple_of(s * OR, 8)
        pltpu.sync_copy(s_acc.at[pl.ds(off, OR)],
                        s_out.at[pl.ds(c * NH + off, OR)])

    @pl.when(s == 7)
    def _():
        pltpu.sync_copy(s_acc.at[pl.ds(7 * OR, NH - 7 * OR)],
                        s_out.at[pl.ds(c * NH + 7 * OR, NH - 7 * OR)])

    @pl.when(jnp.logical_and(s >= 8, s < 15))
    def _():
        off = pl.multiple_of((s - 8) * OR, 8)
        pltpu.sync_copy(b_acc.at[pl.ds(off, OR)],
                        b_out.at[pl.ds(c * NH + off, OR)])

    @pl.when(s == 15)
    def _():
        pltpu.sync_copy(b_acc.at[pl.ds(7 * OR, NH - 7 * OR)],
                        b_out.at[pl.ds(c * NH + 7 * OR, NH - 7 * OR)])


def _tc_body(s_ref, b_ref, nh_ref, wl_ref, wr_ref, wt_ref, bias_ref, cnt_ref,
             o_ref):
    cs = jnp.dot(s_ref[...], wl_ref[...], preferred_element_type=jnp.float32)
    cs = cs + jnp.dot(b_ref[...], wr_ref[...] - wl_ref[...],
                      preferred_element_type=jnp.float32)
    cs = cs + jnp.dot(nh_ref[...], wt_ref[...],
                      preferred_element_type=jnp.float32)
    act = jnp.maximum(cs + bias_ref[...], 0.0)
    o_ref[...] = jnp.where(cnt_ref[...] > 0.0, act, 0.0)


_tc_update = pl.pallas_call(
    _tc_body,
    grid=(N // BN,),
    in_specs=[
        pl.BlockSpec((BN, X), lambda i: (i, 0)),
        pl.BlockSpec((BN, X), lambda i: (i, 0)),
        pl.BlockSpec((BN, X), lambda i: (i, 0)),
        pl.BlockSpec((X, H), lambda i: (0, 0)),
        pl.BlockSpec((X, H), lambda i: (0, 0)),
        pl.BlockSpec((X, H), lambda i: (0, 0)),
        pl.BlockSpec((1, H), lambda i: (0, 0)),
        pl.BlockSpec((BN, 1), lambda i: (i, 0)),
    ],
    out_specs=pl.BlockSpec((BN, H), lambda i: (i, 0)),
    out_shape=jax.ShapeDtypeStruct((N, H), jnp.float32),
)


def kernel(h, nodes_h, edge_index, W_left, W_right, W_top, b_conv):
    src = edge_index[0]
    dst = edge_index[1]
    E = src.shape[0]

    counts = jnp.bincount(dst, length=N)
    starts = jnp.cumsum(counts) - counts
    cnt_f = counts.astype(jnp.float32)
    st_f = starts.astype(jnp.float32)

    zpad = jnp.zeros((2 * K,), jnp.int32)
    src_p = jnp.concatenate([src, zpad])
    dst_p = jnp.concatenate([dst, zpad])

    mid = jnp.searchsorted(dst, NH).astype(jnp.int32)
    los = jnp.stack([jnp.int32(0), mid])
    his = jnp.stack([mid, jnp.int32(E)])
    s_arr = jnp.arange(NS, dtype=jnp.int32)
    rows = []
    for ci in range(NC):
        lo, hi = los[ci], his[ci]
        span = hi - lo
        cpt = (span + NS - 1) // NS
        t_lo = lo + jnp.minimum(s_arr * cpt, span)
        t_hi = lo + jnp.minimum((s_arr + 1) * cpt, span)
        a_lo = (t_lo // 8) * 8
        nch = (t_hi - a_lo + K - 1) // K
        zero = jnp.zeros_like(t_lo)
        rows.append(jnp.stack([a_lo, t_lo, t_hi, nch] + [zero] * 12, axis=1))
    bounds = jnp.concatenate(rows, axis=0).astype(jnp.int32)

    S, B = _sc_segment_sums(h, src_p, dst_p, cnt_f, st_f, bounds)

    return _tc_update(S, B, nodes_h, W_left, W_right, W_top, b_conv,
                      cnt_f[:, None])


# speedup vs baseline: 1.6429x; 1.6429x over previous
import functools

import jax
import jax.numpy as jnp
from jax import lax
from jax.experimental import pallas as pl
from jax.experimental.pallas import tpu as pltpu
from jax.experimental.pallas import tpu_sc as plsc

N = 10000
X = 128
H = 128
K = 128
NC = 2
NS = 16
NH = N // NC
ACC_ROWS = 5008
BN = 1000


_mesh = plsc.VectorSubcoreMesh(core_axis_name="c", subcore_axis_name="s")


@functools.partial(
    pl.kernel,
    out_type=[
        jax.ShapeDtypeStruct((N, X), jnp.float32),
        jax.ShapeDtypeStruct((N, X), jnp.float32),
    ],
    mesh=_mesh,
    scratch_types=[
        pltpu.VMEM((16,), jnp.int32),
        pltpu.VMEM((K,), jnp.int32),
        pltpu.VMEM((K,), jnp.int32),
        pltpu.VMEM((K,), jnp.int32),
        pltpu.VMEM((K,), jnp.int32),
        pltpu.VMEM((K,), jnp.int32),
        pltpu.VMEM((K,), jnp.float32),
        pltpu.VMEM((K,), jnp.float32),
        pltpu.VMEM((K,), jnp.float32),
        pltpu.VMEM((K,), jnp.float32),
        pltpu.VMEM((K, X), jnp.float32),
        pltpu.VMEM((K, X), jnp.float32),
        pltpu.VMEM((K, X), jnp.float32),
        pltpu.VMEM_SHARED((ACC_ROWS, X), jnp.float32),
        pltpu.VMEM_SHARED((ACC_ROWS, X), jnp.float32),
        pltpu.SemaphoreType.DMA,
        pltpu.SemaphoreType.DMA,
        pltpu.SemaphoreType.DMA,
        pltpu.SemaphoreType.DMA,
    ],
)
def _sc_segment_sums(h_hbm, src_hbm, dst_hbm, cnt_hbm, st_hbm, bounds_hbm,
                     s_out, b_out,
                     brow_v, sidx0, sidx1, didx0, didx1, dloc_v,
                     cnt0, cnt1, st0, st1, rows0, rows1, scaled_v,
                     s_acc, b_acc, sem_i0, sem_i1, sem_g0, sem_g1):
    c = lax.axis_index("c")
    s = lax.axis_index("s")
    wid = c * NS + s
    sidx = (sidx0, sidx1)
    didx = (didx0, didx1)
    cnt = (cnt0, cnt1)
    st = (st0, st1)
    rows = (rows0, rows1)
    sem_i = (sem_i0, sem_i1)
    sem_g = (sem_g0, sem_g1)

    zeros16 = jnp.zeros((16,), jnp.float32)
    for r in range(K):
        for j in range(X // 16):
            scaled_v[r, pl.ds(j * 16, 16)] = zeros16
    NZS = ACC_ROWS // K
    for q in range(3):
        zidx = s * 3 + q

        @pl.when(zidx < NZS)
        def _():
            pltpu.async_copy(scaled_v, s_acc.at[pl.ds(zidx * K, K)], sem_g0)
            pltpu.async_copy(scaled_v, b_acc.at[pl.ds(zidx * K, K)], sem_g0)
    for q in range(3):
        zidx = s * 3 + q

        @pl.when(zidx < NZS)
        def _():
            pltpu.make_async_copy(h_hbm.at[pl.ds(0, K)], scaled_v,
                                  sem_g0).wait()
            pltpu.make_async_copy(h_hbm.at[pl.ds(0, K)], scaled_v,
                                  sem_g0).wait()

    @pl.when(s == 0)
    def _():
        pltpu.sync_copy(scaled_v.at[pl.ds(0, ACC_ROWS - NZS * K)],
                        s_acc.at[pl.ds(NZS * K, ACC_ROWS - NZS * K)])
        pltpu.sync_copy(scaled_v.at[pl.ds(0, ACC_ROWS - NZS * K)],
                        b_acc.at[pl.ds(NZS * K, ACC_ROWS - NZS * K)])

    plsc.subcore_barrier()

    pltpu.sync_copy(bounds_hbm.at[wid], brow_v)
    b16 = brow_v[...]
    a_lo = b16[0]
    t_lo = b16[1]
    t_hi = b16[2]
    nch = b16[3]

    def issue_idx(chunk, b):
        base = pl.multiple_of(a_lo + chunk * K, 8)
        pltpu.async_copy(src_hbm.at[pl.ds(base, K)], sidx[b], sem_i[b])
        pltpu.async_copy(dst_hbm.at[pl.ds(base, K)], didx[b], sem_i[b])

    def wait_idx(b):
        pltpu.make_async_copy(src_hbm.at[pl.ds(0, K)], sidx[b],
                              sem_i[b]).wait()
        pltpu.make_async_copy(dst_hbm.at[pl.ds(0, K)], didx[b],
                              sem_i[b]).wait()

    def issue_gathers(b):
        pltpu.async_copy(h_hbm.at[sidx[b]], rows[b], sem_g[b])
        pltpu.async_copy(cnt_hbm.at[didx[b]], cnt[b], sem_g[b])
        pltpu.async_copy(st_hbm.at[didx[b]], st[b], sem_g[b])

    def wait_gathers(b):
        pltpu.make_async_copy(h_hbm.at[pl.ds(0, K)], rows[b],
                              sem_g[b]).wait()
        pltpu.make_async_copy(cnt_hbm.at[pl.ds(0, K)], cnt[b],
                              sem_g[b]).wait()
        pltpu.make_async_copy(st_hbm.at[pl.ds(0, K)], st[b],
                              sem_g[b]).wait()

    @pl.when(nch >= 1)
    def _():
        issue_idx(0, 0)

    @pl.when(nch >= 2)
    def _():
        issue_idx(1, 1)

    @pl.when(nch >= 1)
    def _():
        wait_idx(0)
        issue_gathers(0)

    def pair_body(it, carry):
        for b in range(2):
            chunk = 2 * it + b

            @pl.when(chunk < nch)
            def _():
                base = pl.multiple_of(a_lo + chunk * K, 8)
                wait_gathers(b)
                for g in range(K // 16):
                    lanes = lax.iota(jnp.int32, 16)
                    evec = base + g * 16 + lanes
                    d16 = didx[b][pl.ds(g * 16, 16)]
                    valid = jnp.logical_and(evec >= t_lo, evec < t_hi)
                    dloc_v[pl.ds(g * 16, 16)] = jnp.where(
                        valid, d16 - c * NH, NH)
                    cnt16 = cnt[b][pl.ds(g * 16, 16)]
                    st16 = st[b][pl.ds(g * 16, 16)]
                    pos = evec.astype(jnp.float32) - st16
                    rw16 = pos / jnp.maximum(cnt16 - 1.0, 1.0)
                    for l in range(16):
                        k = g * 16 + l
                        rwb = jnp.full((16,), rw16[l], jnp.float32)
                        for j in range(X // 16):
                            scaled_v[k, pl.ds(j * 16, 16)] = (
                                rwb * rows[b][k, pl.ds(j * 16, 16)])

                @pl.when(chunk + 2 < nch)
                def _():
                    issue_idx(chunk + 2, b)

                pltpu.sync_copy(rows[b], s_acc.at[dloc_v], add=True)
                pltpu.sync_copy(scaled_v, b_acc.at[dloc_v], add=True)

                @pl.when(chunk + 1 < nch)
                def _():
                    wait_idx(1 - b)
                    issue_gathers(1 - b)

        return carry

    lax.fori_loop(0, (nch + 1) // 2, pair_body, 0)
    plsc.subcore_barrier()

    OR = 624

    @pl.when(s < 7)
    def _():
        off = pl.multiple_of(s * OR, 8)
        pltpu.sync_copy(s_acc.at[pl.ds(off, OR)],
                        s_out.at[pl.ds(c * NH + off, OR)])

    @pl.when(s == 7)
    def _():
        pltpu.sync_copy(s_acc.at[pl.ds(7 * OR, NH - 7 * OR)],
                        s_out.at[pl.ds(c * NH + 7 * OR, NH - 7 * OR)])

    @pl.when(jnp.logical_and(s >= 8, s < 15))
    def _():
        off = pl.multiple_of((s - 8) * OR, 8)
        pltpu.sync_copy(b_acc.at[pl.ds(off, OR)],
                        b_out.at[pl.ds(c * NH + off, OR)])

    @pl.when(s == 15)
    def _():
        pltpu.sync_copy(b_acc.at[pl.ds(7 * OR, NH - 7 * OR)],
                        b_out.at[pl.ds(c * NH + 7 * OR, NH - 7 * OR)])


def _tc_body(s_ref, b_ref, nh_ref, wl_ref, wr_ref, wt_ref, bias_ref, cnt_ref,
             o_ref):
    cs = jnp.dot(s_ref[...], wl_ref[...], preferred_element_type=jnp.float32)
    cs = cs + jnp.dot(b_ref[...], wr_ref[...] - wl_ref[...],
                      preferred_element_type=jnp.float32)
    cs = cs + jnp.dot(nh_ref[...], wt_ref[...],
                      preferred_element_type=jnp.float32)
    act = jnp.maximum(cs + bias_ref[...], 0.0)
    o_ref[...] = jnp.where(cnt_ref[...] > 0.0, act, 0.0)


_tc_update = pl.pallas_call(
    _tc_body,
    grid=(N // BN,),
    in_specs=[
        pl.BlockSpec((BN, X), lambda i: (i, 0)),
        pl.BlockSpec((BN, X), lambda i: (i, 0)),
        pl.BlockSpec((BN, X), lambda i: (i, 0)),
        pl.BlockSpec((X, H), lambda i: (0, 0)),
        pl.BlockSpec((X, H), lambda i: (0, 0)),
        pl.BlockSpec((X, H), lambda i: (0, 0)),
        pl.BlockSpec((1, H), lambda i: (0, 0)),
        pl.BlockSpec((BN, 1), lambda i: (i, 0)),
    ],
    out_specs=pl.BlockSpec((BN, H), lambda i: (i, 0)),
    out_shape=jax.ShapeDtypeStruct((N, H), jnp.float32),
)


def kernel(h, nodes_h, edge_index, W_left, W_right, W_top, b_conv):
    src = edge_index[0]
    dst = edge_index[1]
    E = src.shape[0]

    counts = jnp.bincount(dst, length=N)
    starts = jnp.cumsum(counts) - counts
    cnt_f = counts.astype(jnp.float32)
    st_f = starts.astype(jnp.float32)

    zpad = jnp.zeros((K,), jnp.int32)
    src_p = jnp.concatenate([src, zpad])
    dst_p = jnp.concatenate([dst, zpad])

    mid = jnp.searchsorted(dst, NH).astype(jnp.int32)
    los = jnp.stack([jnp.int32(0), mid])
    his = jnp.stack([mid, jnp.int32(E)])
    s_arr = jnp.arange(NS, dtype=jnp.int32)
    rows = []
    for ci in range(NC):
        lo, hi = los[ci], his[ci]
        span = hi - lo
        cpt = (span + NS - 1) // NS
        t_lo = lo + jnp.minimum(s_arr * cpt, span)
        t_hi = lo + jnp.minimum((s_arr + 1) * cpt, span)
        a_lo = (t_lo // 8) * 8
        nch = (t_hi - a_lo + K - 1) // K
        zero = jnp.zeros_like(t_lo)
        rows.append(jnp.stack([a_lo, t_lo, t_hi, nch] + [zero] * 12, axis=1))
    bounds = jnp.concatenate(rows, axis=0).astype(jnp.int32)

    S, B = _sc_segment_sums(h, src_p, dst_p, cnt_f, st_f, bounds)

    return _tc_update(S, B, nodes_h, W_left, W_right, W_top, b_conv,
                      cnt_f[:, None])


# speedup vs baseline: 1.7085x; 1.0399x over previous
import functools

import jax
import jax.numpy as jnp
from jax import lax
from jax.experimental import pallas as pl
from jax.experimental.pallas import tpu as pltpu
from jax.experimental.pallas import tpu_sc as plsc

N = 10000
X = 128
H = 128
K = 128
NC = 2
NS = 16
NH = N // NC
ACC_ROWS = 5008
BN = 1000


_mesh = plsc.VectorSubcoreMesh(core_axis_name="c", subcore_axis_name="s")


@functools.partial(
    pl.kernel,
    out_type=[
        jax.ShapeDtypeStruct((N, X), jnp.float32),
        jax.ShapeDtypeStruct((N, X), jnp.float32),
    ],
    mesh=_mesh,
    scratch_types=[
        pltpu.VMEM((16,), jnp.int32),
        pltpu.VMEM((K,), jnp.int32),
        pltpu.VMEM((K,), jnp.int32),
        pltpu.VMEM((K,), jnp.int32),
        pltpu.VMEM((K,), jnp.int32),
        pltpu.VMEM((K,), jnp.int32),
        pltpu.VMEM((K,), jnp.float32),
        pltpu.VMEM((K,), jnp.float32),
        pltpu.VMEM((K, X), jnp.float32),
        pltpu.VMEM((K, X), jnp.float32),
        pltpu.VMEM((K, X), jnp.float32),
        pltpu.VMEM_SHARED((ACC_ROWS, X), jnp.float32),
        pltpu.VMEM_SHARED((ACC_ROWS, X), jnp.float32),
        pltpu.SemaphoreType.DMA,
        pltpu.SemaphoreType.DMA,
        pltpu.SemaphoreType.DMA,
        pltpu.SemaphoreType.DMA,
    ],
)
def _sc_segment_sums(h_hbm, src_hbm, dst_hbm, st_hbm, bounds_hbm,
                     s_out, b_out,
                     brow_v, sidx0, sidx1, didx0, didx1, dloc_v,
                     st0, st1, rows0, rows1, scaled_v,
                     s_acc, b_acc, sem_i0, sem_i1, sem_g0, sem_g1):
    c = lax.axis_index("c")
    s = lax.axis_index("s")
    wid = c * NS + s
    sidx = (sidx0, sidx1)
    didx = (didx0, didx1)
    st = (st0, st1)
    rows = (rows0, rows1)
    sem_i = (sem_i0, sem_i1)
    sem_g = (sem_g0, sem_g1)

    zeros16 = jnp.zeros((16,), jnp.float32)
    for r in range(K):
        for j in range(X // 16):
            scaled_v[r, pl.ds(j * 16, 16)] = zeros16
    NZS = ACC_ROWS // K
    for q in range(3):
        zidx = s * 3 + q

        @pl.when(zidx < NZS)
        def _():
            pltpu.async_copy(scaled_v, s_acc.at[pl.ds(zidx * K, K)], sem_g0)
            pltpu.async_copy(scaled_v, b_acc.at[pl.ds(zidx * K, K)], sem_g0)
    for q in range(3):
        zidx = s * 3 + q

        @pl.when(zidx < NZS)
        def _():
            pltpu.make_async_copy(h_hbm.at[pl.ds(0, K)], scaled_v,
                                  sem_g0).wait()
            pltpu.make_async_copy(h_hbm.at[pl.ds(0, K)], scaled_v,
                                  sem_g0).wait()

    @pl.when(s == 0)
    def _():
        pltpu.sync_copy(scaled_v.at[pl.ds(0, ACC_ROWS - NZS * K)],
                        s_acc.at[pl.ds(NZS * K, ACC_ROWS - NZS * K)])
        pltpu.sync_copy(scaled_v.at[pl.ds(0, ACC_ROWS - NZS * K)],
                        b_acc.at[pl.ds(NZS * K, ACC_ROWS - NZS * K)])

    plsc.subcore_barrier()

    pltpu.sync_copy(bounds_hbm.at[wid], brow_v)
    b16 = brow_v[...]
    a_lo = b16[0]
    t_lo = b16[1]
    t_hi = b16[2]
    nch = b16[3]

    def issue_idx(chunk, b):
        base = pl.multiple_of(a_lo + chunk * K, 8)
        pltpu.async_copy(src_hbm.at[pl.ds(base, K)], sidx[b], sem_i[b])
        pltpu.async_copy(dst_hbm.at[pl.ds(base, K)], didx[b], sem_i[b])

    def wait_idx(b):
        pltpu.make_async_copy(src_hbm.at[pl.ds(0, K)], sidx[b],
                              sem_i[b]).wait()
        pltpu.make_async_copy(dst_hbm.at[pl.ds(0, K)], didx[b],
                              sem_i[b]).wait()

    def issue_gathers(b):
        pltpu.async_copy(h_hbm.at[sidx[b]], rows[b], sem_g[b])
        pltpu.async_copy(st_hbm.at[didx[b]], st[b], sem_g[b])

    def wait_gathers(b):
        pltpu.make_async_copy(h_hbm.at[pl.ds(0, K)], rows[b],
                              sem_g[b]).wait()
        pltpu.make_async_copy(st_hbm.at[pl.ds(0, K)], st[b],
                              sem_g[b]).wait()

    @pl.when(nch >= 1)
    def _():
        issue_idx(0, 0)

    @pl.when(nch >= 2)
    def _():
        issue_idx(1, 1)

    @pl.when(nch >= 1)
    def _():
        wait_idx(0)
        issue_gathers(0)

    def pair_body(it, carry):
        for b in range(2):
            chunk = 2 * it + b

            @pl.when(chunk < nch)
            def _():
                base = pl.multiple_of(a_lo + chunk * K, 8)
                wait_gathers(b)
                for g in range(K // 16):
                    lanes = lax.iota(jnp.int32, 16)
                    evec = base + g * 16 + lanes
                    d16 = didx[b][pl.ds(g * 16, 16)]
                    valid = jnp.logical_and(evec >= t_lo, evec < t_hi)
                    dloc_v[pl.ds(g * 16, 16)] = jnp.where(
                        valid, d16 - c * NH, NH)
                    st16 = st[b][pl.ds(g * 16, 16)]
                    pos = evec.astype(jnp.float32) - st16
                    for l in range(16):
                        k = g * 16 + l
                        rwb = jnp.full((16,), pos[l], jnp.float32)
                        for j in range(X // 16):
                            scaled_v[k, pl.ds(j * 16, 16)] = (
                                rwb * rows[b][k, pl.ds(j * 16, 16)])

                @pl.when(chunk + 2 < nch)
                def _():
                    issue_idx(chunk + 2, b)

                pltpu.sync_copy(rows[b], s_acc.at[dloc_v], add=True)
                pltpu.sync_copy(scaled_v, b_acc.at[dloc_v], add=True)

                @pl.when(chunk + 1 < nch)
                def _():
                    wait_idx(1 - b)
                    issue_gathers(1 - b)

        return carry

    lax.fori_loop(0, (nch + 1) // 2, pair_body, 0)
    plsc.subcore_barrier()

    OR = 624

    @pl.when(s < 7)
    def _():
        off = pl.multiple_of(s * OR, 8)
        pltpu.sync_copy(s_acc.at[pl.ds(off, OR)],
                        s_out.at[pl.ds(c * NH + off, OR)])

    @pl.when(s == 7)
    def _():
        pltpu.sync_copy(s_acc.at[pl.ds(7 * OR, NH - 7 * OR)],
                        s_out.at[pl.ds(c * NH + 7 * OR, NH - 7 * OR)])

    @pl.when(jnp.logical_and(s >= 8, s < 15))
    def _():
        off = pl.multiple_of((s - 8) * OR, 8)
        pltpu.sync_copy(b_acc.at[pl.ds(off, OR)],
                        b_out.at[pl.ds(c * NH + off, OR)])

    @pl.when(s == 15)
    def _():
        pltpu.sync_copy(b_acc.at[pl.ds(7 * OR, NH - 7 * OR)],
                        b_out.at[pl.ds(c * NH + 7 * OR, NH - 7 * OR)])


def _tc_body(s_ref, b_ref, nh_ref, wl_ref, wr_ref, wt_ref, bias_ref, cnt_ref,
             o_ref):
    cs = jnp.dot(s_ref[...], wl_ref[...], preferred_element_type=jnp.float32)
    b = b_ref[...] / jnp.maximum(cnt_ref[...] - 1.0, 1.0)
    cs = cs + jnp.dot(b, wr_ref[...] - wl_ref[...],
                      preferred_element_type=jnp.float32)
    cs = cs + jnp.dot(nh_ref[...], wt_ref[...],
                      preferred_element_type=jnp.float32)
    act = jnp.maximum(cs + bias_ref[...], 0.0)
    o_ref[...] = jnp.where(cnt_ref[...] > 0.0, act, 0.0)


_tc_update = pl.pallas_call(
    _tc_body,
    grid=(N // BN,),
    in_specs=[
        pl.BlockSpec((BN, X), lambda i: (i, 0)),
        pl.BlockSpec((BN, X), lambda i: (i, 0)),
        pl.BlockSpec((BN, X), lambda i: (i, 0)),
        pl.BlockSpec((X, H), lambda i: (0, 0)),
        pl.BlockSpec((X, H), lambda i: (0, 0)),
        pl.BlockSpec((X, H), lambda i: (0, 0)),
        pl.BlockSpec((1, H), lambda i: (0, 0)),
        pl.BlockSpec((BN, 1), lambda i: (i, 0)),
    ],
    out_specs=pl.BlockSpec((BN, H), lambda i: (i, 0)),
    out_shape=jax.ShapeDtypeStruct((N, H), jnp.float32),
)


def kernel(h, nodes_h, edge_index, W_left, W_right, W_top, b_conv):
    src = edge_index[0]
    dst = edge_index[1]
    E = src.shape[0]

    counts = jnp.bincount(dst, length=N)
    starts = jnp.cumsum(counts) - counts
    cnt_f = counts.astype(jnp.float32)
    st_f = starts.astype(jnp.float32)

    zpad = jnp.zeros((K,), jnp.int32)
    src_p = jnp.concatenate([src, zpad])
    dst_p = jnp.concatenate([dst, zpad])

    mid = jnp.searchsorted(dst, NH).astype(jnp.int32)
    los = jnp.stack([jnp.int32(0), mid])
    his = jnp.stack([mid, jnp.int32(E)])
    s_arr = jnp.arange(NS, dtype=jnp.int32)
    rows = []
    for ci in range(NC):
        lo, hi = los[ci], his[ci]
        span = hi - lo
        cpt = (span + NS - 1) // NS
        t_lo = lo + jnp.minimum(s_arr * cpt, span)
        t_hi = lo + jnp.minimum((s_arr + 1) * cpt, span)
        a_lo = (t_lo // 8) * 8
        nch = (t_hi - a_lo + K - 1) // K
        zero = jnp.zeros_like(t_lo)
        rows.append(jnp.stack([a_lo, t_lo, t_hi, nch] + [zero] * 12, axis=1))
    bounds = jnp.concatenate(rows, axis=0).astype(jnp.int32)

    S, B = _sc_segment_sums(h, src_p, dst_p, st_f, bounds)

    return _tc_update(S, B, nodes_h, W_left, W_right, W_top, b_conv,
                      cnt_f[:, None])


# speedup vs baseline: 2.0646x; 1.2084x over previous
import functools

import jax
import jax.numpy as jnp
from jax import lax
from jax.experimental import pallas as pl
from jax.experimental.pallas import tpu as pltpu
from jax.experimental.pallas import tpu_sc as plsc

N = 10000
X = 128
H = 128
K = 128
NC = 2
NS = 16
NH = N // NC
ACC_ROWS = 5008
BN = 1000


_mesh = plsc.VectorSubcoreMesh(core_axis_name="c", subcore_axis_name="s")


@functools.partial(
    pl.kernel,
    out_type=[
        jax.ShapeDtypeStruct((N, X), jnp.float32),
        jax.ShapeDtypeStruct((N, X), jnp.float32),
    ],
    mesh=_mesh,
    scratch_types=[
        pltpu.VMEM((16,), jnp.int32),
        pltpu.VMEM((K,), jnp.int32),
        pltpu.VMEM((K,), jnp.int32),
        pltpu.VMEM((K,), jnp.int32),
        pltpu.VMEM((K,), jnp.int32),
        pltpu.VMEM((K,), jnp.int32),
        pltpu.VMEM((K,), jnp.int32),
        pltpu.VMEM((K,), jnp.float32),
        pltpu.VMEM((K,), jnp.float32),
        pltpu.VMEM((K, X), jnp.float32),
        pltpu.VMEM((K, X), jnp.float32),
        pltpu.VMEM((K, X), jnp.float32),
        pltpu.VMEM_SHARED((ACC_ROWS, X), jnp.float32),
        pltpu.VMEM_SHARED((ACC_ROWS, X), jnp.float32),
        pltpu.SemaphoreType.DMA,
        pltpu.SemaphoreType.DMA,
        pltpu.SemaphoreType.DMA,
        pltpu.SemaphoreType.DMA,
        pltpu.SemaphoreType.DMA,
        pltpu.SemaphoreType.DMA,
        pltpu.SemaphoreType.DMA,
    ],
)
def _sc_segment_sums(h_hbm, src_hbm, dst_hbm, st_hbm, bounds_hbm,
                     s_out, b_out,
                     brow_v, sidx0, sidx1, didx0, didx1, dloc0, dloc1,
                     st0, st1, rows0, rows1, scaled0,
                     s_acc, b_acc, sem_i0, sem_i1, sem_g0, sem_g1,
                     sem_s0, sem_s1, sem_b0):
    c = lax.axis_index("c")
    s = lax.axis_index("s")
    wid = c * NS + s
    sidx = (sidx0, sidx1)
    didx = (didx0, didx1)
    dloc = (dloc0, dloc1)
    st = (st0, st1)
    rows = (rows0, rows1)
    sem_i = (sem_i0, sem_i1)
    sem_g = (sem_g0, sem_g1)
    sem_s = (sem_s0, sem_s1)

    zeros16 = jnp.zeros((16,), jnp.float32)
    for r in range(K):
        for j in range(X // 16):
            scaled0[r, pl.ds(j * 16, 16)] = zeros16
    NZS = ACC_ROWS // K
    for q in range(3):
        zidx = s * 3 + q

        @pl.when(zidx < NZS)
        def _():
            pltpu.async_copy(scaled0, s_acc.at[pl.ds(zidx * K, K)], sem_g0)
            pltpu.async_copy(scaled0, b_acc.at[pl.ds(zidx * K, K)], sem_g0)
    for q in range(3):
        zidx = s * 3 + q

        @pl.when(zidx < NZS)
        def _():
            pltpu.make_async_copy(h_hbm.at[pl.ds(0, K)], scaled0,
                                  sem_g0).wait()
            pltpu.make_async_copy(h_hbm.at[pl.ds(0, K)], scaled0,
                                  sem_g0).wait()

    @pl.when(s == 0)
    def _():
        pltpu.sync_copy(scaled0.at[pl.ds(0, ACC_ROWS - NZS * K)],
                        s_acc.at[pl.ds(NZS * K, ACC_ROWS - NZS * K)])
        pltpu.sync_copy(scaled0.at[pl.ds(0, ACC_ROWS - NZS * K)],
                        b_acc.at[pl.ds(NZS * K, ACC_ROWS - NZS * K)])

    plsc.subcore_barrier()

    pltpu.sync_copy(bounds_hbm.at[wid], brow_v)
    b16 = brow_v[...]
    a_lo = b16[0]
    t_lo = b16[1]
    t_hi = b16[2]
    nch = b16[3]

    def issue_idx(chunk, b):
        base = pl.multiple_of(a_lo + chunk * K, 8)
        pltpu.async_copy(src_hbm.at[pl.ds(base, K)], sidx[b], sem_i[b])
        pltpu.async_copy(dst_hbm.at[pl.ds(base, K)], didx[b], sem_i[b])

    def wait_idx(b):
        pltpu.make_async_copy(src_hbm.at[pl.ds(0, K)], sidx[b],
                              sem_i[b]).wait()
        pltpu.make_async_copy(dst_hbm.at[pl.ds(0, K)], didx[b],
                              sem_i[b]).wait()

    def issue_gathers(b):
        pltpu.async_copy(h_hbm.at[sidx[b]], rows[b], sem_g[b])
        pltpu.async_copy(st_hbm.at[didx[b]], st[b], sem_g[b])

    def wait_gathers(b):
        pltpu.make_async_copy(h_hbm.at[pl.ds(0, K)], rows[b],
                              sem_g[b]).wait()
        pltpu.make_async_copy(st_hbm.at[pl.ds(0, K)], st[b],
                              sem_g[b]).wait()

    @pl.when(nch >= 1)
    def _():
        issue_idx(0, 0)

    @pl.when(nch >= 2)
    def _():
        issue_idx(1, 1)

    @pl.when(nch >= 1)
    def _():
        wait_idx(0)
        issue_gathers(0)

    def pair_body(it, carry):
        for b in range(2):
            chunk = 2 * it + b

            @pl.when(chunk < nch)
            def _():
                base = pl.multiple_of(a_lo + chunk * K, 8)
                wait_gathers(b)

                for g in range(K // 16):
                    lanes = lax.iota(jnp.int32, 16)
                    evec = base + g * 16 + lanes
                    d16 = didx[b][pl.ds(g * 16, 16)]
                    valid = jnp.logical_and(evec >= t_lo, evec < t_hi)
                    dloc[b][pl.ds(g * 16, 16)] = jnp.where(
                        valid, d16 - c * NH, NH)

                pltpu.async_copy(rows[b], s_acc.at[dloc[b]], sem_s[b],
                                 add=True)

                @pl.when(chunk >= 1)
                def _():
                    pltpu.make_async_copy(scaled0, b_acc.at[pl.ds(0, K)],
                                          sem_b0).wait()

                for g in range(K // 16):
                    lanes = lax.iota(jnp.int32, 16)
                    evec = base + g * 16 + lanes
                    st16 = st[b][pl.ds(g * 16, 16)]
                    pos = evec.astype(jnp.float32) - st16
                    for l in range(16):
                        k = g * 16 + l
                        rwb = jnp.full((16,), pos[l], jnp.float32)
                        for j in range(X // 16):
                            scaled0[k, pl.ds(j * 16, 16)] = (
                                rwb * rows[b][k, pl.ds(j * 16, 16)])

                @pl.when(chunk + 2 < nch)
                def _():
                    issue_idx(chunk + 2, b)

                pltpu.async_copy(scaled0, b_acc.at[dloc[b]], sem_b0,
                                 add=True)

                @pl.when(chunk + 1 < nch)
                def _():
                    @pl.when(chunk >= 1)
                    def _():
                        pltpu.make_async_copy(rows[1 - b],
                                              s_acc.at[pl.ds(0, K)],
                                              sem_s[1 - b]).wait()
                    wait_idx(1 - b)
                    issue_gathers(1 - b)

        return carry

    lax.fori_loop(0, (nch + 1) // 2, pair_body, 0)

    for b in range(2):
        @pl.when(nch >= b + 1)
        def _():
            pltpu.make_async_copy(rows[b], s_acc.at[pl.ds(0, K)],
                                  sem_s[b]).wait()

    @pl.when(nch >= 1)
    def _():
        pltpu.make_async_copy(scaled0, b_acc.at[pl.ds(0, K)],
                              sem_b0).wait()

    plsc.subcore_barrier()

    OR = 624

    @pl.when(s < 7)
    def _():
        off = pl.multiple_of(s * OR, 8)
        pltpu.sync_copy(s_acc.at[pl.ds(off, OR)],
                        s_out.at[pl.ds(c * NH + off, OR)])

    @pl.when(s == 7)
    def _():
        pltpu.sync_copy(s_acc.at[pl.ds(7 * OR, NH - 7 * OR)],
                        s_out.at[pl.ds(c * NH + 7 * OR, NH - 7 * OR)])

    @pl.when(jnp.logical_and(s >= 8, s < 15))
    def _():
        off = pl.multiple_of((s - 8) * OR, 8)
        pltpu.sync_copy(b_acc.at[pl.ds(off, OR)],
                        b_out.at[pl.ds(c * NH + off, OR)])

    @pl.when(s == 15)
    def _():
        pltpu.sync_copy(b_acc.at[pl.ds(7 * OR, NH - 7 * OR)],
                        b_out.at[pl.ds(c * NH + 7 * OR, NH - 7 * OR)])


def _tc_body(s_ref, b_ref, nh_ref, wl_ref, wr_ref, wt_ref, bias_ref, cnt_ref,
             o_ref):
    cs = jnp.dot(s_ref[...], wl_ref[...], preferred_element_type=jnp.float32)
    b = b_ref[...] / jnp.maximum(cnt_ref[...] - 1.0, 1.0)
    cs = cs + jnp.dot(b, wr_ref[...] - wl_ref[...],
                      preferred_element_type=jnp.float32)
    cs = cs + jnp.dot(nh_ref[...], wt_ref[...],
                      preferred_element_type=jnp.float32)
    act = jnp.maximum(cs + bias_ref[...], 0.0)
    o_ref[...] = jnp.where(cnt_ref[...] > 0.0, act, 0.0)


_tc_update = pl.pallas_call(
    _tc_body,
    grid=(N // BN,),
    in_specs=[
        pl.BlockSpec((BN, X), lambda i: (i, 0)),
        pl.BlockSpec((BN, X), lambda i: (i, 0)),
        pl.BlockSpec((BN, X), lambda i: (i, 0)),
        pl.BlockSpec((X, H), lambda i: (0, 0)),
        pl.BlockSpec((X, H), lambda i: (0, 0)),
        pl.BlockSpec((X, H), lambda i: (0, 0)),
        pl.BlockSpec((1, H), lambda i: (0, 0)),
        pl.BlockSpec((BN, 1), lambda i: (i, 0)),
    ],
    out_specs=pl.BlockSpec((BN, H), lambda i: (i, 0)),
    out_shape=jax.ShapeDtypeStruct((N, H), jnp.float32),
)


def kernel(h, nodes_h, edge_index, W_left, W_right, W_top, b_conv):
    src = edge_index[0]
    dst = edge_index[1]
    E = src.shape[0]

    counts = jnp.bincount(dst, length=N)
    starts = jnp.cumsum(counts) - counts
    cnt_f = counts.astype(jnp.float32)
    st_f = starts.astype(jnp.float32)

    zpad = jnp.zeros((K,), jnp.int32)
    src_p = jnp.concatenate([src, zpad])
    dst_p = jnp.concatenate([dst, zpad])

    mid = jnp.searchsorted(dst, NH).astype(jnp.int32)
    los = jnp.stack([jnp.int32(0), mid])
    his = jnp.stack([mid, jnp.int32(E)])
    s_arr = jnp.arange(NS, dtype=jnp.int32)
    rows = []
    for ci in range(NC):
        lo, hi = los[ci], his[ci]
        span = hi - lo
        cpt = (span + NS - 1) // NS
        t_lo = lo + jnp.minimum(s_arr * cpt, span)
        t_hi = lo + jnp.minimum((s_arr + 1) * cpt, span)
        a_lo = (t_lo // 8) * 8
        nch = (t_hi - a_lo + K - 1) // K
        zero = jnp.zeros_like(t_lo)
        rows.append(jnp.stack([a_lo, t_lo, t_hi, nch] + [zero] * 12, axis=1))
    bounds = jnp.concatenate(rows, axis=0).astype(jnp.int32)

    S, B = _sc_segment_sums(h, src_p, dst_p, st_f, bounds)

    return _tc_update(S, B, nodes_h, W_left, W_right, W_top, b_conv,
                      cnt_f[:, None])
